# baseline (speedup 1.0000x reference)
import jax
import jax.numpy as jnp
from jax import lax
from jax.experimental import pallas as pl
from jax.experimental.pallas import tpu as pltpu
from jax.experimental.pallas import tpu_sc as plsc

D = 32
B = 16384
L = 200
NC = 2
NS = 16
LANES = 16
NW = NC * NS
TOK = B * L
TOK_PER_TILE = TOK // NW
B_PER_TILE = B // NW
SW = 4 * L
SPP = 16
PANELTOK = SW * SPP
NPANEL = TOK_PER_TILE // PANELTOK
ROWS_PER_PANEL = PANELTOK // L
UNROLL = 8
SCALE = 1.0 / L


def _embed_body(tokens_hbm, table_hbm, out_hbm,
                idx_v, buf0, buf1, outbuf, sem0, sem1):
    c = lax.axis_index("c")
    s = lax.axis_index("s")
    tile = c * NS + s
    tok0 = tile * TOK_PER_TILE
    out_row0 = tile * B_PER_TILE

    vzero = jnp.zeros((LANES,), jnp.float32)

    def start_gather(g, buf, sem):
        pltpu.async_copy(table_hbm.at[idx_v.at[pl.ds(g * SW, SW)]], buf, sem)

    def wait_gather(buf, sem):
        pltpu.make_async_copy(table_hbm.at[idx_v.at[pl.ds(0, SW)]], buf, sem).wait()

    def accumulate(buf, row0):
        for q in range(SW // L):
            def body(i, carry):
                a0, a1 = carry
                for u in range(UNROLL):
                    r = q * L + i * UNROLL + u
                    a0 = a0 + buf[r, pl.ds(0, LANES)]
                    a1 = a1 + buf[r, pl.ds(LANES, LANES)]
                return (a0, a1)

            a0, a1 = lax.fori_loop(0, L // UNROLL, body, (vzero, vzero))
            outbuf[row0 + q, pl.ds(0, LANES)] = a0 * SCALE
            outbuf[row0 + q, pl.ds(LANES, LANES)] = a1 * SCALE

    @pl.loop(0, NPANEL)
    def _(p):
        pltpu.sync_copy(tokens_hbm.at[pl.ds(tok0 + p * PANELTOK, PANELTOK)], idx_v)
        start_gather(0, buf0, sem0)

        @pl.loop(0, SPP, step=2)
        def _(g):
            start_gather(g + 1, buf1, sem1)
            wait_gather(buf0, sem0)
            accumulate(buf0, p * ROWS_PER_PANEL + g * (SW // L))

            @pl.when(g + 2 < SPP)
            def _():
                start_gather(g + 2, buf0, sem0)

            wait_gather(buf1, sem1)
            accumulate(buf1, p * ROWS_PER_PANEL + (g + 1) * (SW // L))

    pltpu.sync_copy(outbuf, out_hbm.at[pl.ds(out_row0, B_PER_TILE)])


@jax.jit
def kernel(tokens, table):
    t32 = tokens.astype(jnp.int32)
    tokens1d = jnp.where(t32 < 0, 0, t32).reshape(TOK)

    mesh = plsc.VectorSubcoreMesh(core_axis_name="c", subcore_axis_name="s")
    run = pl.kernel(
        _embed_body,
        out_type=jax.ShapeDtypeStruct((B, D), jnp.float32),
        mesh=mesh,
        compiler_params=pltpu.CompilerParams(use_tc_tiling_on_sc=False),
        scratch_types=[
            pltpu.VMEM((PANELTOK,), jnp.int32),
            pltpu.VMEM((SW, D), jnp.float32),
            pltpu.VMEM((SW, D), jnp.float32),
            pltpu.VMEM((B_PER_TILE, D), jnp.float32),
            pltpu.SemaphoreType.DMA,
            pltpu.SemaphoreType.DMA,
        ],
    )
    return run(tokens1d, table)
